# baseline (speedup 1.0000x reference)
import jax
import jax.numpy as jnp
from jax import lax
from jax.experimental import pallas as pl
from jax.experimental.pallas import tpu as pltpu

B = 256
IN_DIM = 512
OUT_DIM = 10000
BJ = 400
NJ = OUT_DIM // BJ


def _gcn_kernel(x_ref, w_ref, adj_ref, bias_ref, out_ref, s_ref):
    @pl.when(pl.program_id(0) == 0)
    def _():
        s_ref[...] = jnp.dot(
            x_ref[...].astype(jnp.bfloat16),
            w_ref[...].astype(jnp.bfloat16),
            preferred_element_type=jnp.float32,
        ).astype(jnp.bfloat16)

    out_ref[...] = (
        lax.dot_general(
            adj_ref[...].astype(jnp.bfloat16),
            s_ref[...],
            (((1,), (1,)), ((), ())),
            preferred_element_type=jnp.float32,
        )
        + bias_ref[...]
    )


def kernel(input, adj, weight, bias):
    outT = pl.pallas_call(
        _gcn_kernel,
        grid=(NJ,),
        in_specs=[
            pl.BlockSpec((B, IN_DIM), lambda j: (0, 0)),
            pl.BlockSpec((IN_DIM, OUT_DIM), lambda j: (0, 0)),
            pl.BlockSpec((BJ, OUT_DIM), lambda j: (j, 0)),
            pl.BlockSpec((BJ, 1), lambda j: (j, 0)),
        ],
        out_specs=pl.BlockSpec((BJ, B), lambda j: (j, 0)),
        out_shape=jax.ShapeDtypeStruct((OUT_DIM, B), jnp.float32),
        scratch_shapes=[pltpu.VMEM((B, OUT_DIM), jnp.bfloat16)],
        compiler_params=pltpu.CompilerParams(
            vmem_limit_bytes=100 * 1024 * 1024,
        ),
    )(input, weight, adj, bias.reshape(OUT_DIM, 1))
    return outT.T


# speedup vs baseline: 1.0086x; 1.0086x over previous
import jax
import jax.numpy as jnp
from jax import lax
from jax.experimental import pallas as pl
from jax.experimental.pallas import tpu as pltpu

B = 256
IN_DIM = 512
OUT_DIM = 10000
BJ = 400
NJ = OUT_DIM // BJ


def _gcn_kernel(x_ref, w_ref, adj_ref, bias_ref, out_ref, sT_ref):
    @pl.when(pl.program_id(0) == 0)
    def _():
        sT_ref[...] = lax.dot_general(
            w_ref[...].astype(jnp.bfloat16),
            x_ref[...].astype(jnp.bfloat16),
            (((0,), (1,)), ((), ())),
            preferred_element_type=jnp.float32,
        ).astype(jnp.bfloat16)

    out_ref[...] = (
        jnp.dot(
            adj_ref[...].astype(jnp.bfloat16),
            sT_ref[...],
            preferred_element_type=jnp.float32,
        )
        + bias_ref[...]
    )


def kernel(input, adj, weight, bias):
    outT = pl.pallas_call(
        _gcn_kernel,
        grid=(NJ,),
        in_specs=[
            pl.BlockSpec((B, IN_DIM), lambda j: (0, 0)),
            pl.BlockSpec((IN_DIM, OUT_DIM), lambda j: (0, 0)),
            pl.BlockSpec((BJ, OUT_DIM), lambda j: (j, 0)),
            pl.BlockSpec((BJ, 1), lambda j: (j, 0)),
        ],
        out_specs=pl.BlockSpec((BJ, B), lambda j: (j, 0)),
        out_shape=jax.ShapeDtypeStruct((OUT_DIM, B), jnp.float32),
        scratch_shapes=[pltpu.VMEM((OUT_DIM, B), jnp.bfloat16)],
        compiler_params=pltpu.CompilerParams(
            vmem_limit_bytes=100 * 1024 * 1024,
        ),
    )(input, weight, adj, bias.reshape(OUT_DIM, 1))
    return outT.T
